# Initial kernel scaffold; baseline (speedup 1.0000x reference)
import jax
import jax.numpy as jnp
from jax.experimental import pallas as pl


def kernel(atoms, pos, edge_index, batch, emb, Wr1, br1, Wr2, br2, U2, U3, alpha, Wp1, bp1, Wp2, bp2):
    raise NotImplementedError("write your pallas kernel here")


# speedup vs baseline: 1.0281x; 1.0281x over previous
import functools

import jax
import jax.numpy as jnp
from jax.experimental import pallas as pl
from jax.experimental.pallas import tpu as pltpu

N = 10000
E = 160000
C = 64
NB = 8
P = 5
R_MAX = 10.0
L = 2
G = 8
M = 9
CM = C * M


def _sph_k(u):
    x, y, z = u[:, 0], u[:, 1], u[:, 2]
    s3 = jnp.sqrt(3.0)
    s5 = jnp.sqrt(5.0)
    s15 = jnp.sqrt(15.0)
    return jnp.stack([
        jnp.ones_like(x),
        s3 * x, s3 * y, s3 * z,
        s15 * x * y, s15 * y * z,
        (s5 / 2.0) * (3.0 * z * z - 1.0),
        s15 * x * z, (s15 / 2.0) * (x * x - y * y)
    ], axis=-1)


def _radial_k(r):
    x = r / R_MAX
    n = jnp.arange(1, NB + 1, dtype=jnp.float32)
    bessel = jnp.sqrt(2.0 / R_MAX) * jnp.sin(n[None, :] * jnp.pi * x[:, None]) / jnp.clip(r, 1e-6)[:, None]
    p = float(P)
    env = (1.0 - ((p + 1.0) * (p + 2.0) / 2.0) * x ** P
           + p * (p + 2.0) * x ** (P + 1)
           - (p * (p + 1.0) / 2.0) * x ** (P + 2))
    env = jnp.where(x < 1.0, env, 0.0)
    return bessel * env[:, None]


def _node_chain_body(f_ref, sc_ref, alpha_ref, rt_ref, ucat_ref, s9_ref,
                     sum9_ref, out_ref, s_ref):
    bn = f_ref.shape[0]
    f = f_ref[...]
    x = jnp.concatenate([f[:, k * M:(k + 1) * M] for k in range(C)], axis=0)
    fi = jnp.dot(x, rt_ref[:, :81], preferred_element_type=jnp.float32)
    fj = jnp.dot(x, rt_ref[:, 81:], preferred_element_type=jnp.float32)
    outer = fi * fj
    gcat = jnp.dot(outer, ucat_ref[...], preferred_element_type=jnp.float32)
    g2 = gcat[:, :M]
    t = gcat[:, M:]
    g3 = jnp.dot(t * fi, s9_ref[...], preferred_element_type=jnp.float32)
    a0 = alpha_ref[0]
    a1 = alpha_ref[1]
    a2 = alpha_ref[2]
    out = a0 * x + a1 * g2 + a2 * g3
    out_flat = jnp.concatenate(
        [out[k * bn:(k + 1) * bn, :] for k in range(C)], axis=1)
    h_new = out_flat + sc_ref[...]
    out_ref[...] = h_new
    s_ref[...] = jnp.dot(h_new, sum9_ref[...],
                         preferred_element_type=jnp.float32) * (1.0 / M)


@functools.partial(jax.jit, static_argnames=("bn",))
def _node_chain(f, sc, alpha_l, rt, ucat, s9, sum9, bn=200):
    grid = N // bn
    return pl.pallas_call(
        _node_chain_body,
        grid=(grid,),
        in_specs=[
            pl.BlockSpec((bn, CM), lambda i: (i, 0)),
            pl.BlockSpec((bn, CM), lambda i: (i, 0)),
            pl.BlockSpec(memory_space=pltpu.SMEM),
            pl.BlockSpec((M, 162), lambda i: (0, 0)),
            pl.BlockSpec((81, 90), lambda i: (0, 0)),
            pl.BlockSpec((81, M), lambda i: (0, 0)),
            pl.BlockSpec((CM, C), lambda i: (0, 0)),
        ],
        out_specs=[
            pl.BlockSpec((bn, CM), lambda i: (i, 0)),
            pl.BlockSpec((bn, C), lambda i: (i, 0)),
        ],
        out_shape=[
            jax.ShapeDtypeStruct((N, CM), jnp.float32),
            jax.ShapeDtypeStruct((N, C), jnp.float32),
        ],
    )(f, sc, alpha_l, rt, ucat, s9, sum9)


def kernel(atoms, pos, edge_index, batch, emb, Wr1, br1, Wr2, br2, U2, U3,
           alpha, Wp1, bp1, Wp2, bp2):
    src, dst = edge_index[0], edge_index[1]
    vec = pos[src] - pos[dst]
    r = jnp.linalg.norm(vec, axis=-1)
    u = vec / jnp.clip(r, 1e-6)[:, None]
    sh = _sph_k(u)
    ef = _radial_k(r)

    i9 = jnp.eye(M, dtype=jnp.float32)
    rmat = jnp.kron(i9, jnp.ones((1, M), jnp.float32))
    tmat = jnp.kron(jnp.ones((1, M), jnp.float32), i9)
    rt = jnp.concatenate([rmat, tmat], axis=1)
    s9 = jnp.kron(jnp.ones((M, 1), jnp.float32), i9)
    sum9 = jnp.kron(jnp.eye(C, dtype=jnp.float32), jnp.ones((M, 1), jnp.float32))

    h = emb[atoms]
    s = h
    for l in range(L):
        w = jax.nn.relu(ef @ Wr1[l] + br1[l]) @ Wr2[l] + br2[l]
        msg = (w * s[src])[:, :, None] * sh[:, None, :]
        f = jax.ops.segment_sum(msg.reshape(E, CM), dst, num_segments=N)
        sc = jnp.pad(h, ((0, 0), (0, CM - h.shape[-1])))
        ucat = jnp.concatenate(
            [U2[l].reshape(81, M), U3[l].reshape(81, 81)], axis=1)
        h, s = _node_chain(f, sc, alpha[l], rt, ucat, s9, sum9)

    hs = h[:, :C]
    pooled = jax.ops.segment_sum(hs, batch, num_segments=G)
    return jax.nn.relu(pooled @ Wp1 + bp1) @ Wp2 + bp2
